# Initial kernel scaffold; baseline (speedup 1.0000x reference)
import jax
import jax.numpy as jnp
from jax.experimental import pallas as pl


def kernel(x, table):
    raise NotImplementedError("write your pallas kernel here")


# speedup vs baseline: 4.8117x; 4.8117x over previous
import functools

import jax
import jax.numpy as jnp
from jax import lax
from jax.experimental import pallas as pl
from jax.experimental.pallas import tpu as pltpu
from jax.experimental.pallas import tpu_sc as plsc

VOCAB = 1000000
D_MODEL = 32

NC = 2
NS = 16
NW = NC * NS

CHUNK = 1024


def _make_lookup(B):
    assert B % (NW * CHUNK) == 0
    b_per_w = B // NW
    n_chunks = b_per_w // CHUNK
    mesh = plsc.VectorSubcoreMesh(core_axis_name="c", subcore_axis_name="s")

    @functools.partial(
        pl.kernel,
        mesh=mesh,
        compiler_params=pltpu.CompilerParams(use_tc_tiling_on_sc=False),
        out_type=jax.ShapeDtypeStruct((B, D_MODEL), jnp.float32),
        scratch_types=[
            pltpu.VMEM((CHUNK,), jnp.int32),
            pltpu.VMEM((CHUNK, D_MODEL), jnp.float32),
            pltpu.SemaphoreType.DMA,
        ],
    )
    def lookup(x_hbm, table_hbm, out_hbm, idx_v, rows_v, sem):
        wid = lax.axis_index("s") * NC + lax.axis_index("c")
        w_base = wid * b_per_w

        @pl.loop(0, n_chunks)
        def _chunk(g):
            base = w_base + g * CHUNK
            pltpu.sync_copy(x_hbm.at[pl.ds(base, CHUNK)], idx_v)
            pltpu.async_copy(table_hbm.at[idx_v], rows_v, sem).wait()
            pltpu.sync_copy(rows_v, out_hbm.at[pl.ds(base, CHUNK)])

    return lookup


def kernel(x, table):
    B = x.shape[0] * x.shape[1]
    flat = x.reshape(B)
    out = _make_lookup(B)(flat, table)
    return out.reshape(x.shape[0], x.shape[1], D_MODEL)


# speedup vs baseline: 5.0260x; 1.0445x over previous
import functools

import jax
import jax.numpy as jnp
from jax import lax
from jax.experimental import pallas as pl
from jax.experimental.pallas import tpu as pltpu
from jax.experimental.pallas import tpu_sc as plsc

VOCAB = 1000000
D_MODEL = 32

NC = 2
NS = 16
NW = NC * NS

CHUNK = 1600
NBUF = 2


def _make_lookup(B):
    assert B % (NW * CHUNK) == 0
    b_per_w = B // NW
    n_chunks = b_per_w // CHUNK
    assert n_chunks % NBUF == 0 and n_chunks >= 2 * NBUF
    n_groups = n_chunks // NBUF
    mesh = plsc.VectorSubcoreMesh(core_axis_name="c", subcore_axis_name="s")

    @functools.partial(
        pl.kernel,
        mesh=mesh,
        compiler_params=pltpu.CompilerParams(use_tc_tiling_on_sc=False),
        out_type=jax.ShapeDtypeStruct((B, D_MODEL), jnp.float32),
        scratch_types=[
            pltpu.VMEM((NBUF, CHUNK), jnp.int32),
            pltpu.VMEM((NBUF, CHUNK, D_MODEL), jnp.float32),
        ] + [pltpu.SemaphoreType.DMA] * (3 * NBUF),
    )
    def lookup(x_hbm, table_hbm, out_hbm, idx_v, rows_v, *sems):
        si = sems[0:NBUF]
        sg = sems[NBUF:2 * NBUF]
        sw = sems[2 * NBUF:3 * NBUF]
        wid = lax.axis_index("s") * NC + lax.axis_index("c")
        w_base = wid * b_per_w

        def start_idx(g, b):
            pltpu.async_copy(
                x_hbm.at[pl.ds(w_base + g * CHUNK, CHUNK)], idx_v.at[b], si[b])

        def wait_idx(b):
            pltpu.make_async_copy(
                x_hbm.at[pl.ds(0, CHUNK)], idx_v.at[b], si[b]).wait()

        def start_gather(b):
            pltpu.async_copy(table_hbm.at[idx_v.at[b]], rows_v.at[b], sg[b])

        def wait_gather(b):
            pltpu.make_async_copy(
                table_hbm.at[idx_v.at[b]], rows_v.at[b], sg[b]).wait()

        def start_wb(g, b):
            pltpu.async_copy(
                rows_v.at[b], out_hbm.at[pl.ds(w_base + g * CHUNK, CHUNK)], sw[b])

        def wait_wb(b):
            pltpu.make_async_copy(
                rows_v.at[b], out_hbm.at[pl.ds(0, CHUNK)], sw[b]).wait()

        for b in range(NBUF):
            start_idx(b, b)
        for b in range(NBUF):
            wait_idx(b)
            start_gather(b)
        for b in range(NBUF):
            wait_gather(b)
            start_wb(b, b)
            start_idx(b + NBUF, b)

        @pl.loop(1, n_groups)
        def _group(h):
            base = h * NBUF
            for b in range(NBUF):
                wait_idx(b)
                wait_wb(b)
                start_gather(b)
            for b in range(NBUF):
                wait_gather(b)
                start_wb(base + b, b)
                g_pref = jnp.minimum(base + NBUF + b, n_chunks - 1)
                start_idx(g_pref, b)

        for b in range(NBUF):
            wait_idx(b)
            wait_wb(b)

    return lookup


def kernel(x, table):
    B = x.shape[0] * x.shape[1]
    flat = x.reshape(B)
    out = _make_lookup(B)(flat, table)
    return out.reshape(x.shape[0], x.shape[1], D_MODEL)
